# Initial kernel scaffold; baseline (speedup 1.0000x reference)
import jax
import jax.numpy as jnp
from jax.experimental import pallas as pl


def kernel(input_x, input_r, e2triple, triple2e, r2triple, emb_table, W_ih, W_hh, b_ih, b_hh, W_lin, b_lin):
    raise NotImplementedError("write your pallas kernel here")


# speedup vs baseline: 1.0517x; 1.0517x over previous
import jax
import jax.numpy as jnp
from jax.experimental import pallas as pl

E_ENT = 100000
B = 128
COL_BLK = 12800


def _onehot_body(x_ref, out_ref):
    j = pl.program_id(0)
    cols = jax.lax.broadcasted_iota(jnp.int32, (B, COL_BLK), 1) + j * COL_BLK
    out_ref[...] = (cols == x_ref[...]).astype(jnp.float32)


def kernel(input_x, input_r, e2triple, triple2e, r2triple, emb_table,
           W_ih, W_hh, b_ih, b_hh, W_lin, b_lin):
    x2d = input_x.astype(jnp.int32).reshape(B, 1)
    grid = (pl.cdiv(E_ENT, COL_BLK),)
    return pl.pallas_call(
        _onehot_body,
        grid=grid,
        in_specs=[pl.BlockSpec((B, 1), lambda j: (0, 0))],
        out_specs=pl.BlockSpec((B, COL_BLK), lambda j: (0, j)),
        out_shape=jax.ShapeDtypeStruct((B, E_ENT), jnp.float32),
    )(x2d)
